# baseline (speedup 1.0000x reference)
import jax
import jax.numpy as jnp
from jax import lax
from jax.experimental import pallas as pl
from jax.experimental.pallas import tpu as pltpu
from jax.experimental.pallas import tpu_sc as plsc

B, T, N, D = 8, 50, 1024, 128
ROWS = B * T * N
NW = 32
ROWS_PER_W = ROWS // NW
R = 128
C = ROWS_PER_W // R
NB = 5
PF = 3
L = 16
GROUPS = C // NB


def _sc_body(x_hbm, emb_hbm, out_hbm, emb_v, b0, b1, b2, b3, b4,
             si0, si1, si2, si3, si4, so0, so1, so2, so3, so4):
    cid = lax.axis_index("c")
    sid = lax.axis_index("s")
    wid = sid * 2 + cid
    base = wid * ROWS_PER_W

    pltpu.sync_copy(emb_hbm, emb_v)

    bufs = (b0, b1, b2, b3, b4)
    sins = (si0, si1, si2, si3, si4)
    souts = (so0, so1, so2, so3, so4)

    def start_in(c, b):
        pltpu.async_copy(x_hbm.at[pl.ds(base + c * R, R), :], bufs[b], sins[b])

    def wait_in(b):
        pltpu.make_async_copy(x_hbm.at[pl.ds(0, R), :], bufs[b], sins[b]).wait()

    def start_out(c, b):
        pltpu.async_copy(bufs[b], out_hbm.at[pl.ds(base + c * R, R), :], souts[b])

    def wait_out(b):
        pltpu.make_async_copy(bufs[b], out_hbm.at[pl.ds(0, R), :], souts[b]).wait()

    def compute(c, b):
        t = ((base + c * R) // N) % T
        regs = [emb_v[t, pl.ds(L * v, L)] for v in range(D // L)]

        def row(r, _):
            for v in range(D // L):
                plsc.addupdate(bufs[b].at[r, pl.ds(L * v, L)], regs[v])
            return 0

        lax.fori_loop(0, R, row, 0, unroll=4)

    def step(c, b, *, guard_out, guard_pf):
        p = c + PF
        pb = (b + PF) % NB
        if guard_out:
            wait_out(pb)
        if guard_pf:
            @pl.when(p < C)
            def _():
                start_in(p, pb)
        else:
            start_in(p, pb)
        wait_in(b)
        compute(c, b)
        start_out(c, b)

    for b in range(PF):
        start_in(jnp.int32(b), b)
    for b in range(NB):
        step(jnp.int32(b), b, guard_out=(b + PF >= NB), guard_pf=False)

    def group(g, _):
        for b in range(NB):
            step(g * NB + b, b, guard_out=True, guard_pf=True)
        return 0

    lax.fori_loop(1, GROUPS, group, 0)

    for c in range(C - (NB - PF), C):
        wait_out(c % NB)


def kernel(x, emb_table):
    xf = x.reshape(ROWS, D)
    mesh = plsc.VectorSubcoreMesh(core_axis_name="c", subcore_axis_name="s")
    out = pl.kernel(
        _sc_body,
        out_type=jax.ShapeDtypeStruct((ROWS, D), jnp.float32),
        mesh=mesh,
        scratch_types=[
            pltpu.VMEM((T, D), jnp.float32),
            *[pltpu.VMEM((R, D), jnp.float32) for _ in range(NB)],
            *[pltpu.SemaphoreType.DMA for _ in range(2 * NB)],
        ],
    )(xf, emb_table)
    return out.reshape(B, T, N, D)
